# Initial kernel scaffold; baseline (speedup 1.0000x reference)
import jax
import jax.numpy as jnp
from jax.experimental import pallas as pl


def kernel(x, s_c1_w, s_c1_b, s_bn1_g, s_bn1_b, s_c2_w, s_c2_b, s_bn2_g, s_bn2_b, s_c3_w, s_c3_b, s_bn3_g, s_bn3_b, s_fc1_w, s_fc1_b, s_bn4_g, s_bn4_b, s_fc2_w, s_fc2_b, s_bn5_g, s_bn5_b, s_fc3_w, s_fc3_b, c1_w, c1_b, bn1_g, bn1_b, c2_w, c2_b, bn2_g, bn2_b, c3_w, c3_b, bn3_g, bn3_b):
    raise NotImplementedError("write your pallas kernel here")


# speedup vs baseline: 1.3913x; 1.3913x over previous
import functools

import jax
import jax.numpy as jnp
from jax import lax
from jax.experimental import pallas as pl
from jax.experimental.pallas import tpu as pltpu

_EPS = 1e-5


def _split_hi_lo(v):
    hi = v.astype(jnp.bfloat16)
    lo = (v - hi.astype(jnp.float32)).astype(jnp.bfloat16)
    return hi, lo


def _dotf(a, wh, wl, hi_prec):
    if not hi_prec:
        return jnp.dot(a.astype(jnp.bfloat16), wh,
                       preferred_element_type=jnp.float32)
    ah, al = _split_hi_lo(a)
    return (jnp.dot(ah, wh, preferred_element_type=jnp.float32)
            + jnp.dot(ah, wl, preferred_element_type=jnp.float32)
            + jnp.dot(al, wh, preferred_element_type=jnp.float32))


def _conv1_relu(x, w1s, shift1):
    h = (x[:, 0:1] * w1s[0:1, :]
         + x[:, 1:2] * w1s[1:2, :]
         + x[:, 2:3] * w1s[2:3, :])
    return jnp.maximum(h + shift1, 0.0)


def _gram(a_bf16):
    return lax.dot_general(a_bf16, a_bf16, (((0,), (0,)), ((), ())),
                           preferred_element_type=jnp.float32)


def _moments_kernel(x_ref, w1s_ref, shift1_ref, cs_ref, gm_ref,
                    *, n_valid, tn):
    r = pl.program_id(1)
    a1 = _conv1_relu(x_ref[0], w1s_ref[0], shift1_ref[...])
    row = r * tn + lax.broadcasted_iota(jnp.int32, (tn, 1), 0)
    a1 = jnp.where(row < n_valid, a1, 0.0)

    @pl.when(r == 0)
    def _init():
        cs_ref[...] = jnp.zeros_like(cs_ref)
        gm_ref[...] = jnp.zeros_like(gm_ref)

    cs_ref[0] += jnp.sum(a1, axis=0, keepdims=True)
    gm_ref[0] += _gram(a1.astype(jnp.bfloat16))


def _feature_kernel(x_ref, w1s_ref, shift1_ref, w2h_ref, w2l_ref, shift2_ref,
                    w3h_ref, w3l_ref, mx_ref, cs_ref, gm_ref,
                    *, n_valid, tn, nb, hi_prec):
    r = pl.program_id(1)
    a1 = _conv1_relu(x_ref[0], w1s_ref[0], shift1_ref[...])
    y2 = _dotf(a1, w2h_ref[...], w2l_ref[...], hi_prec)
    a2 = jnp.maximum(y2 + shift2_ref[...], 0.0)
    row = r * tn + lax.broadcasted_iota(jnp.int32, (tn, 1), 0)
    valid = row < n_valid
    a2 = jnp.where(valid, a2, 0.0)
    y3 = _dotf(a2, w3h_ref[...], w3l_ref[...], hi_prec)

    @pl.when(r == 0)
    def _init():
        cs_ref[...] = jnp.zeros_like(cs_ref)
        gm_ref[...] = jnp.zeros_like(gm_ref)
        mx_ref[...] = jnp.full(mx_ref.shape, -jnp.inf, mx_ref.dtype)

    cs_ref[0] += jnp.sum(a2, axis=0, keepdims=True)
    gm_ref[0] += _gram(a2.astype(jnp.bfloat16))

    @pl.when(r < nb - 1)
    def _full():
        mx_ref[0] = jnp.maximum(mx_ref[0], jnp.max(y3, axis=0, keepdims=True))

    @pl.when(r == nb - 1)
    def _tail():
        mx_ref[0] = jnp.maximum(
            mx_ref[0],
            jnp.max(jnp.where(valid, y3, -jnp.inf), axis=0, keepdims=True))


def _head_kernel(g_ref, w1_ref, g4_ref, b4_ref, w2_ref, g5_ref, b5_ref,
                 w3_ref, b3_ref, o_ref):
    def bn_relu(h, gamma, beta):
        m = jnp.mean(h, axis=0, keepdims=True)
        d = h - m
        v = jnp.mean(d * d, axis=0, keepdims=True)
        return jnp.maximum(d * (gamma * lax.rsqrt(v + _EPS)) + beta, 0.0)

    hp = lax.Precision.HIGHEST
    h = jnp.dot(g_ref[...], w1_ref[...], preferred_element_type=jnp.float32,
                precision=hp)
    h = bn_relu(h, g4_ref[...], b4_ref[...])
    h = jnp.dot(h, w2_ref[...], preferred_element_type=jnp.float32,
                precision=hp)
    h = bn_relu(h, g5_ref[...], b5_ref[...])
    o_ref[...] = (jnp.dot(h, w3_ref[...], preferred_element_type=jnp.float32,
                          precision=hp) + b3_ref[...])


def _bn_affine(total_sum, total_ssq, gamma, beta, nrows):
    mean = total_sum.reshape(1, -1) / nrows
    var = jnp.maximum(total_ssq.reshape(1, -1) / nrows - mean * mean, 0.0)
    scale = gamma.reshape(1, -1) * lax.rsqrt(var + _EPS)
    shift = beta.reshape(1, -1) - mean * scale
    return scale, shift


def _quad_diag(w, gram):
    return jnp.einsum('ck,cd,dk->k', w, gram, w, precision='highest')


def _point_stack(xp, colsum_x, gram_x, w1_b, w2, w3, bn1, bn2, bn3,
                 *, n_valid, tn, final_relu, hi_prec):
    B, n_pad, _ = xp.shape
    C1, C2, C3 = w1_b.shape[-1], w2.shape[1], w3.shape[1]
    nb = n_pad // tn
    nrows = B * n_valid
    g1, be1 = bn1
    g2, be2 = bn2
    g3, be3 = bn3

    if w1_b.shape[0] == 1:
        sum1 = jnp.einsum('c,ck->k', jnp.sum(colsum_x, 0), w1_b[0],
                          precision='highest')
        ssq1 = _quad_diag(w1_b[0], jnp.sum(gram_x, 0))
    else:
        sum1 = jnp.einsum('bc,bck->k', colsum_x, w1_b, precision='highest')
        ssq1 = jnp.einsum('bck,bcd,bdk->k', w1_b, gram_x, w1_b,
                          precision='highest')
    scale1, shift1 = _bn_affine(sum1, ssq1, g1, be1, nrows)
    w1s = w1_b * scale1[None]

    cparams = pltpu.CompilerParams(
        dimension_semantics=("parallel", "arbitrary"))
    x_spec = pl.BlockSpec((1, tn, 3), lambda b, r: (b, r, 0))
    w1_spec = pl.BlockSpec((1, 3, C1),
                           (lambda b, r: (b, 0, 0)) if w1s.shape[0] > 1
                           else (lambda b, r: (0, 0, 0)))

    def bcast(a):
        return pl.BlockSpec(a.shape, lambda b, r: (0,) * a.ndim)

    def acc(shape):
        return pl.BlockSpec((1,) + shape, lambda b, r: (b, 0, 0))

    cs1, gm1 = pl.pallas_call(
        functools.partial(_moments_kernel, n_valid=n_valid, tn=tn),
        out_shape=(jax.ShapeDtypeStruct((B, 1, C1), jnp.float32),
                   jax.ShapeDtypeStruct((B, C1, C1), jnp.float32)),
        grid=(B, nb),
        in_specs=[x_spec, w1_spec, bcast(shift1)],
        out_specs=(acc((1, C1)), acc((C1, C1))),
        compiler_params=cparams,
    )(xp, w1s, shift1)

    sum2 = jnp.einsum('c,ck->k', jnp.sum(cs1, (0, 1)), w2, precision='highest')
    ssq2 = _quad_diag(w2, jnp.sum(gm1, 0))
    scale2, shift2 = _bn_affine(sum2, ssq2, g2, be2, nrows)
    w2h, w2l = _split_hi_lo(w2 * scale2)
    w3h, w3l = _split_hi_lo(w3 * jnp.where(g3 >= 0.0, 1.0, -1.0))

    mx, cs2, gm2 = pl.pallas_call(
        functools.partial(_feature_kernel, n_valid=n_valid, tn=tn, nb=nb,
                          hi_prec=hi_prec),
        out_shape=(jax.ShapeDtypeStruct((B, 1, C3), jnp.float32),
                   jax.ShapeDtypeStruct((B, 1, C2), jnp.float32),
                   jax.ShapeDtypeStruct((B, C2, C2), jnp.float32)),
        grid=(B, nb),
        in_specs=[x_spec, w1_spec, bcast(shift1), bcast(w2h), bcast(w2l),
                  bcast(shift2), bcast(w3h), bcast(w3l)],
        out_specs=(acc((1, C3)), acc((1, C2)), acc((C2, C2))),
        compiler_params=cparams,
    )(xp, w1s, shift1, w2h, w2l, shift2, w3h, w3l)

    sum3 = jnp.einsum('c,ck->k', jnp.sum(cs2, (0, 1)), w3, precision='highest')
    ssq3 = _quad_diag(w3, jnp.sum(gm2, 0))
    scale3, shift3 = _bn_affine(sum3, ssq3, g3, be3, nrows)

    gf = mx.reshape(B, C3) * jnp.abs(scale3) + shift3
    return jnp.maximum(gf, 0.0) if final_relu else gf


def kernel(x, s_c1_w, s_c1_b, s_bn1_g, s_bn1_b, s_c2_w, s_c2_b, s_bn2_g,
           s_bn2_b, s_c3_w, s_c3_b, s_bn3_g, s_bn3_b, s_fc1_w, s_fc1_b,
           s_bn4_g, s_bn4_b, s_fc2_w, s_fc2_b, s_bn5_g, s_bn5_b, s_fc3_w,
           s_fc3_b, c1_w, c1_b, bn1_g, bn1_b, c2_w, c2_b, bn2_g, bn2_b,
           c3_w, c3_b, bn3_g, bn3_b):
    B, _, N = x.shape
    tn = 512 if N >= 512 else max(8, -(-N // 8) * 8)
    n_pad = -(-N // tn) * tn

    xp = jnp.transpose(x, (0, 2, 1))
    if n_pad != N:
        xp = jnp.pad(xp, ((0, 0), (0, n_pad - N), (0, 0)))

    colsum_x = jnp.sum(x, axis=2)
    gram_x = jnp.einsum('bcn,bdn->bcd', x, x, precision='highest')

    gf_stn = _point_stack(
        xp, colsum_x, gram_x, s_c1_w[None], s_c2_w, s_c3_w,
        (s_bn1_g, s_bn1_b), (s_bn2_g, s_bn2_b), (s_bn3_g, s_bn3_b),
        n_valid=N, tn=tn, final_relu=True, hi_prec=True)

    b3_eff = s_fc3_b + jnp.eye(3, dtype=jnp.float32).reshape(1, 9)
    head_in = (gf_stn, s_fc1_w, s_bn4_g, s_bn4_b,
               s_fc2_w, s_bn5_g, s_bn5_b, s_fc3_w, b3_eff)
    trans_flat = pl.pallas_call(
        _head_kernel,
        out_shape=jax.ShapeDtypeStruct((B, 9), jnp.float32),
        grid=(1,),
        in_specs=[pl.BlockSpec(a.shape, lambda i, _nd=a.ndim: (0,) * _nd)
                  for a in head_in],
        out_specs=pl.BlockSpec((B, 9), lambda i: (0, 0)),
        compiler_params=pltpu.CompilerParams(
            dimension_semantics=("arbitrary",)),
    )(*head_in)
    trans = trans_flat.reshape(B, 3, 3)

    w1_eff = jnp.einsum('bij,jk->bik', trans, c1_w, precision='highest')
    gf = _point_stack(
        xp, colsum_x, gram_x, w1_eff, c2_w, c3_w,
        (bn1_g, bn1_b), (bn2_g, bn2_b), (bn3_g, bn3_b),
        n_valid=N, tn=tn, final_relu=False, hi_prec=False)

    return gf, trans, None
